# Initial kernel scaffold; baseline (speedup 1.0000x reference)
import jax
import jax.numpy as jnp
from jax.experimental import pallas as pl


def kernel(x, edge_index, batch, params):
    raise NotImplementedError("write your pallas kernel here")


# speedup vs baseline: 5.6468x; 5.6468x over previous
import functools

import jax
import jax.numpy as jnp
from jax import lax
from jax.experimental import pallas as pl
from jax.experimental.pallas import tpu as pltpu
from jax.experimental.pallas import tpu_sc as plsc

N = 10000
E = 320000
H = 128
C = 10
G = 128
BN_EPS = 1e-5

NC = 2
NS = 16
NW = NC * NS

CHUNK = 128
TOTAL_CHUNKS = E // CHUNK
ITERS = (TOTAL_CHUNKS + NW - 1) // NW
ROWS_PER_TILE = 624
TAIL_ROWS = N - NS * ROWS_PER_TILE


def _segsum_body(h_hbm, src_hbm, dst_hbm, zeros_hbm, out_hbm,
                 src_v, dst_v, rows_v, acc_sh, sem):
    cid = lax.axis_index("c")
    sid = lax.axis_index("s")
    wid = sid * NC + cid

    row0 = sid * ROWS_PER_TILE
    pltpu.sync_copy(zeros_hbm.at[pl.ds(row0, ROWS_PER_TILE)],
                    acc_sh.at[pl.ds(row0, ROWS_PER_TILE)])

    @pl.when(sid == NS - 1)
    def _():
        pltpu.sync_copy(zeros_hbm.at[pl.ds(NS * ROWS_PER_TILE, TAIL_ROWS)],
                        acc_sh.at[pl.ds(NS * ROWS_PER_TILE, TAIL_ROWS)])

    plsc.subcore_barrier()

    def body(i, _):
        chunk = wid + i * NW

        @pl.when(chunk < TOTAL_CHUNKS)
        def _():
            base = chunk * CHUNK
            pltpu.sync_copy(src_hbm.at[pl.ds(base, CHUNK)], src_v)
            pltpu.sync_copy(dst_hbm.at[pl.ds(base, CHUNK)], dst_v)
            pltpu.async_copy(h_hbm.at[src_v], rows_v, sem).wait()
            pltpu.sync_copy(rows_v, acc_sh.at[dst_v], add=True)

        return ()

    lax.fori_loop(0, ITERS, body, (), unroll=False)
    plsc.subcore_barrier()

    pltpu.sync_copy(acc_sh.at[pl.ds(row0, ROWS_PER_TILE)],
                    out_hbm.at[cid].at[pl.ds(row0, ROWS_PER_TILE)])

    @pl.when(sid == NS - 1)
    def _():
        pltpu.sync_copy(acc_sh.at[pl.ds(NS * ROWS_PER_TILE, TAIL_ROWS)],
                        out_hbm.at[cid].at[pl.ds(NS * ROWS_PER_TILE, TAIL_ROWS)])


@functools.cache
def _get_segsum():
    return functools.partial(
        pl.kernel,
        out_type=jax.ShapeDtypeStruct((NC, N, H), jnp.float32),
        mesh=plsc.VectorSubcoreMesh(core_axis_name="c", subcore_axis_name="s",
                                    num_cores=NC, num_subcores=NS),
        scratch_types=[
            pltpu.VMEM((CHUNK,), jnp.int32),
            pltpu.VMEM((CHUNK,), jnp.int32),
            pltpu.VMEM((CHUNK, H), jnp.float32),
            pltpu.VMEM_SHARED((N, H), jnp.float32),
            pltpu.SemaphoreType.DMA,
        ],
    )(_segsum_body)


def _bn(z, g, b):
    m = jnp.mean(z, axis=0, keepdims=True)
    v = jnp.mean((z - m) * (z - m), axis=0, keepdims=True)
    return (z - m) * lax.rsqrt(v + BN_EPS) * g + b


def _dense_body(h_ref, a_ref, eps_ref, w1_ref, b1_ref, w2_ref, b2_ref,
                g1_ref, be1_ref, g2_ref, be2_ref, out_ref):
    h = h_ref[...]
    aggr = a_ref[0] + a_ref[1]
    z = (1.0 + eps_ref[0, 0]) * h + aggr
    z = jnp.maximum(
        jnp.dot(z, w1_ref[...], preferred_element_type=jnp.float32)
        + b1_ref[...], 0.0)
    z = jnp.maximum(
        jnp.dot(z, w2_ref[...], preferred_element_type=jnp.float32)
        + b2_ref[...], 0.0)
    z = _bn(z, g1_ref[...], be1_ref[...])
    z = _bn(z, g2_ref[...], be2_ref[...])
    out_ref[...] = z


_dense = pl.pallas_call(
    _dense_body,
    out_shape=jax.ShapeDtypeStruct((N, H), jnp.float32),
)


def _head_body(h_ref, batch_ref, w1_ref, b1_ref, w2_ref, b2_ref, out_ref):
    h = h_ref[...]
    b = batch_ref[...]
    gids = lax.broadcasted_iota(jnp.int32, (N, G), 1)
    onehot = jnp.where(b == gids, 1.0, 0.0)
    sums = lax.dot_general(onehot, h, (((0,), (0,)), ((), ())),
                           preferred_element_type=jnp.float32)
    counts = jnp.sum(onehot, axis=0)[:, None]
    pooled = sums / jnp.maximum(counts, 1.0)
    z = jnp.maximum(
        jnp.dot(pooled, w1_ref[...], preferred_element_type=jnp.float32)
        + b1_ref[...], 0.0)
    z = (jnp.dot(z, w2_ref[...], preferred_element_type=jnp.float32)
         + b2_ref[...])
    m = jnp.max(z, axis=-1, keepdims=True)
    lse = jnp.log(jnp.sum(jnp.exp(z - m), axis=-1, keepdims=True))
    out_ref[...] = z - m - lse


_head = pl.pallas_call(
    _head_body,
    out_shape=jax.ShapeDtypeStruct((G, C), jnp.float32),
)


def kernel(x, edge_index, batch, params):
    src = edge_index[0].astype(jnp.int32)
    dst = edge_index[1].astype(jnp.int32)
    zeros = jnp.zeros((N, H), jnp.float32)
    batch2d = batch.astype(jnp.int32).reshape(N, 1)

    h = x
    for l in range(3):
        p = params['convs'][l]
        bn = params['bns'][l]
        partials = _get_segsum()(h, src, dst, zeros)
        h = _dense(h, partials,
                   p['eps'].reshape(1, 1),
                   p['W1'], p['b1'].reshape(1, H),
                   p['W2'], p['b2'].reshape(1, H),
                   p['g'].reshape(1, H), p['be'].reshape(1, H),
                   bn['g'].reshape(1, H), bn['be'].reshape(1, H))

    logits = _head(h, batch2d,
                   params['lin1_W'], params['lin1_b'].reshape(1, H),
                   params['lin2_W'], params['lin2_b'].reshape(1, C))
    bit_sum = jnp.zeros((1,), jnp.float32)
    return (logits, bit_sum)


# speedup vs baseline: 8.6483x; 1.5315x over previous
import functools

import jax
import jax.numpy as jnp
from jax import lax
from jax.experimental import pallas as pl
from jax.experimental.pallas import tpu as pltpu
from jax.experimental.pallas import tpu_sc as plsc

N = 10000
E = 320000
H = 128
C = 10
G = 128
BN_EPS = 1e-5

NC = 2
NS = 16
NW = NC * NS

CHUNK = 128
TOTAL_CHUNKS = E // CHUNK
MAIN_ITERS = TOTAL_CHUNKS // NW
TAIL_CHUNKS = TOTAL_CHUNKS - MAIN_ITERS * NW
ROWS_PER_TILE = 624
TAIL_ROWS = N - NS * ROWS_PER_TILE


def _segsum_body(h_hbm, src_hbm, dst_hbm, zeros_hbm, out_hbm,
                 src_v, dst_v, rows_v, acc_sh, sems):
    cid = lax.axis_index("c")
    sid = lax.axis_index("s")
    wid = sid * NC + cid

    row0 = sid * ROWS_PER_TILE
    pltpu.sync_copy(zeros_hbm.at[pl.ds(row0, ROWS_PER_TILE)],
                    acc_sh.at[pl.ds(row0, ROWS_PER_TILE)])

    @pl.when(sid == NS - 1)
    def _():
        pltpu.sync_copy(zeros_hbm.at[pl.ds(NS * ROWS_PER_TILE, TAIL_ROWS)],
                        acc_sh.at[pl.ds(NS * ROWS_PER_TILE, TAIL_ROWS)])

    plsc.subcore_barrier()

    def load_and_fire(i, b):
        base = (wid + i * NW) * CHUNK
        pltpu.sync_copy(src_hbm.at[pl.ds(base, CHUNK)], src_v.at[b])
        pltpu.sync_copy(dst_hbm.at[pl.ds(base, CHUNK)], dst_v.at[b])
        pltpu.async_copy(h_hbm.at[src_v.at[b]], rows_v.at[b], sems.at[b])

    load_and_fire(0, 0)

    def body(it, _):
        for b in range(2):
            i = it * 2 + b
            if b == 0:
                load_and_fire(i + 1, 1)
            else:
                @pl.when(it < MAIN_ITERS // 2 - 1)
                def _():
                    load_and_fire(i + 1, 0)
            pltpu.make_async_copy(h_hbm.at[src_v.at[b]], rows_v.at[b],
                                  sems.at[b]).wait()
            pltpu.sync_copy(rows_v.at[b], acc_sh.at[dst_v.at[b]], add=True)
        return ()

    lax.fori_loop(0, MAIN_ITERS // 2, body, (), unroll=False)

    @pl.when(wid < TAIL_CHUNKS)
    def _():
        base = (MAIN_ITERS * NW + wid) * CHUNK
        pltpu.sync_copy(src_hbm.at[pl.ds(base, CHUNK)], src_v.at[0])
        pltpu.sync_copy(dst_hbm.at[pl.ds(base, CHUNK)], dst_v.at[0])
        pltpu.async_copy(h_hbm.at[src_v.at[0]], rows_v.at[0], sems.at[0]).wait()
        pltpu.sync_copy(rows_v.at[0], acc_sh.at[dst_v.at[0]], add=True)

    plsc.subcore_barrier()

    pltpu.sync_copy(acc_sh.at[pl.ds(row0, ROWS_PER_TILE)],
                    out_hbm.at[cid].at[pl.ds(row0, ROWS_PER_TILE)])

    @pl.when(sid == NS - 1)
    def _():
        pltpu.sync_copy(acc_sh.at[pl.ds(NS * ROWS_PER_TILE, TAIL_ROWS)],
                        out_hbm.at[cid].at[pl.ds(NS * ROWS_PER_TILE, TAIL_ROWS)])


@functools.cache
def _get_segsum():
    return functools.partial(
        pl.kernel,
        out_type=jax.ShapeDtypeStruct((NC, N, H), jnp.float32),
        mesh=plsc.VectorSubcoreMesh(core_axis_name="c", subcore_axis_name="s",
                                    num_cores=NC, num_subcores=NS),
        scratch_types=[
            pltpu.VMEM((2, CHUNK), jnp.int32),
            pltpu.VMEM((2, CHUNK), jnp.int32),
            pltpu.VMEM((2, CHUNK, H), jnp.float32),
            pltpu.VMEM_SHARED((N, H), jnp.float32),
            pltpu.SemaphoreType.DMA((2,)),
        ],
    )(_segsum_body)


def _bn(z, g, b):
    m = jnp.mean(z, axis=0, keepdims=True)
    v = jnp.mean((z - m) * (z - m), axis=0, keepdims=True)
    return (z - m) * lax.rsqrt(v + BN_EPS) * g + b


def _dense_body(h_ref, a_ref, eps_ref, w1_ref, b1_ref, w2_ref, b2_ref,
                g1_ref, be1_ref, g2_ref, be2_ref, out_ref):
    h = h_ref[...]
    aggr = a_ref[0] + a_ref[1]
    z = (1.0 + eps_ref[0, 0]) * h + aggr
    z = jnp.maximum(
        jnp.dot(z, w1_ref[...], preferred_element_type=jnp.float32)
        + b1_ref[...], 0.0)
    z = jnp.maximum(
        jnp.dot(z, w2_ref[...], preferred_element_type=jnp.float32)
        + b2_ref[...], 0.0)
    z = _bn(z, g1_ref[...], be1_ref[...])
    z = _bn(z, g2_ref[...], be2_ref[...])
    out_ref[...] = z


_dense = pl.pallas_call(
    _dense_body,
    out_shape=jax.ShapeDtypeStruct((N, H), jnp.float32),
)


def _head_body(h_ref, batch_ref, w1_ref, b1_ref, w2_ref, b2_ref, out_ref):
    h = h_ref[...]
    b = batch_ref[...]
    gids = lax.broadcasted_iota(jnp.int32, (N, G), 1)
    onehot = jnp.where(b == gids, 1.0, 0.0)
    sums = lax.dot_general(onehot, h, (((0,), (0,)), ((), ())),
                           preferred_element_type=jnp.float32)
    counts = jnp.sum(onehot, axis=0)[:, None]
    pooled = sums / jnp.maximum(counts, 1.0)
    z = jnp.maximum(
        jnp.dot(pooled, w1_ref[...], preferred_element_type=jnp.float32)
        + b1_ref[...], 0.0)
    z = (jnp.dot(z, w2_ref[...], preferred_element_type=jnp.float32)
         + b2_ref[...])
    m = jnp.max(z, axis=-1, keepdims=True)
    lse = jnp.log(jnp.sum(jnp.exp(z - m), axis=-1, keepdims=True))
    out_ref[...] = z - m - lse


_head = pl.pallas_call(
    _head_body,
    out_shape=jax.ShapeDtypeStruct((G, C), jnp.float32),
)


def kernel(x, edge_index, batch, params):
    src = edge_index[0].astype(jnp.int32)
    dst = edge_index[1].astype(jnp.int32)
    zeros = jnp.zeros((N, H), jnp.float32)
    batch2d = batch.astype(jnp.int32).reshape(N, 1)

    h = x
    for l in range(3):
        p = params['convs'][l]
        bn = params['bns'][l]
        partials = _get_segsum()(h, src, dst, zeros)
        h = _dense(h, partials,
                   p['eps'].reshape(1, 1),
                   p['W1'], p['b1'].reshape(1, H),
                   p['W2'], p['b2'].reshape(1, H),
                   p['g'].reshape(1, H), p['be'].reshape(1, H),
                   bn['g'].reshape(1, H), bn['be'].reshape(1, H))

    logits = _head(h, batch2d,
                   params['lin1_W'], params['lin1_b'].reshape(1, H),
                   params['lin2_W'], params['lin2_b'].reshape(1, C))
    bit_sum = jnp.zeros((1,), jnp.float32)
    return (logits, bit_sum)


# speedup vs baseline: 8.6640x; 1.0018x over previous
import functools

import jax
import jax.numpy as jnp
from jax import lax
from jax.experimental import pallas as pl
from jax.experimental.pallas import tpu as pltpu
from jax.experimental.pallas import tpu_sc as plsc

N = 10000
E = 320000
H = 128
C = 10
G = 128
BN_EPS = 1e-5

NC = 2
NS = 16
NW = NC * NS

CHUNK = 128
TOTAL_CHUNKS = E // CHUNK
MAIN_ITERS = TOTAL_CHUNKS // NW
TAIL_CHUNKS = TOTAL_CHUNKS - MAIN_ITERS * NW
ROWS_PER_TILE = 624
TAIL_ROWS = N - NS * ROWS_PER_TILE


def _segsum_body(h_hbm, src_hbm, dst_hbm, zeros_hbm, out_hbm,
                 src_v, dst_v, rows_v, acc_sh, g_sems, s_sems):
    cid = lax.axis_index("c")
    sid = lax.axis_index("s")
    wid = sid * NC + cid

    row0 = sid * ROWS_PER_TILE
    pltpu.sync_copy(zeros_hbm.at[pl.ds(row0, ROWS_PER_TILE)],
                    acc_sh.at[pl.ds(row0, ROWS_PER_TILE)])

    @pl.when(sid == NS - 1)
    def _():
        pltpu.sync_copy(zeros_hbm.at[pl.ds(NS * ROWS_PER_TILE, TAIL_ROWS)],
                        acc_sh.at[pl.ds(NS * ROWS_PER_TILE, TAIL_ROWS)])

    plsc.subcore_barrier()

    def load_and_fire(i, b):
        base = (wid + i * NW) * CHUNK
        pltpu.sync_copy(src_hbm.at[pl.ds(base, CHUNK)], src_v.at[b])
        pltpu.sync_copy(dst_hbm.at[pl.ds(base, CHUNK)], dst_v.at[b])
        pltpu.async_copy(h_hbm.at[src_v.at[b]], rows_v.at[b], g_sems.at[b])

    def wait_gather(b):
        pltpu.make_async_copy(h_hbm.at[src_v.at[b]], rows_v.at[b],
                              g_sems.at[b]).wait()

    def fire_scatter(b):
        pltpu.async_copy(rows_v.at[b], acc_sh.at[dst_v.at[b]], s_sems.at[b],
                         add=True)

    def wait_scatter(b):
        pltpu.make_async_copy(rows_v.at[b], acc_sh.at[dst_v.at[b]],
                              s_sems.at[b]).wait()

    load_and_fire(0, 0)

    def body(it, _):
        for b in range(2):
            i = it * 2 + b
            nb = 1 - b
            if b == 0:
                @pl.when(it > 0)
                def _():
                    wait_scatter(nb)
                load_and_fire(i + 1, nb)
            else:
                @pl.when(it < MAIN_ITERS // 2 - 1)
                def _():
                    wait_scatter(nb)
                    load_and_fire(i + 1, nb)
            wait_gather(b)
            fire_scatter(b)
        return ()

    lax.fori_loop(0, MAIN_ITERS // 2, body, (), unroll=False)
    wait_scatter(0)
    wait_scatter(1)

    @pl.when(wid < TAIL_CHUNKS)
    def _():
        base = (MAIN_ITERS * NW + wid) * CHUNK
        pltpu.sync_copy(src_hbm.at[pl.ds(base, CHUNK)], src_v.at[0])
        pltpu.sync_copy(dst_hbm.at[pl.ds(base, CHUNK)], dst_v.at[0])
        pltpu.async_copy(h_hbm.at[src_v.at[0]], rows_v.at[0], g_sems.at[0]).wait()
        pltpu.sync_copy(rows_v.at[0], acc_sh.at[dst_v.at[0]], add=True)

    plsc.subcore_barrier()

    pltpu.sync_copy(acc_sh.at[pl.ds(row0, ROWS_PER_TILE)],
                    out_hbm.at[cid].at[pl.ds(row0, ROWS_PER_TILE)])

    @pl.when(sid == NS - 1)
    def _():
        pltpu.sync_copy(acc_sh.at[pl.ds(NS * ROWS_PER_TILE, TAIL_ROWS)],
                        out_hbm.at[cid].at[pl.ds(NS * ROWS_PER_TILE, TAIL_ROWS)])


@functools.cache
def _get_segsum():
    return functools.partial(
        pl.kernel,
        out_type=jax.ShapeDtypeStruct((NC, N, H), jnp.float32),
        mesh=plsc.VectorSubcoreMesh(core_axis_name="c", subcore_axis_name="s",
                                    num_cores=NC, num_subcores=NS),
        scratch_types=[
            pltpu.VMEM((2, CHUNK), jnp.int32),
            pltpu.VMEM((2, CHUNK), jnp.int32),
            pltpu.VMEM((2, CHUNK, H), jnp.float32),
            pltpu.VMEM_SHARED((N, H), jnp.float32),
            pltpu.SemaphoreType.DMA((2,)),
            pltpu.SemaphoreType.DMA((2,)),
        ],
    )(_segsum_body)


def _bn(z, g, b):
    m = jnp.mean(z, axis=0, keepdims=True)
    v = jnp.mean((z - m) * (z - m), axis=0, keepdims=True)
    return (z - m) * lax.rsqrt(v + BN_EPS) * g + b


def _dense_body(h_ref, a_ref, eps_ref, w1_ref, b1_ref, w2_ref, b2_ref,
                g1_ref, be1_ref, g2_ref, be2_ref, out_ref):
    h = h_ref[...]
    aggr = a_ref[0] + a_ref[1]
    z = (1.0 + eps_ref[0, 0]) * h + aggr
    z = jnp.maximum(
        jnp.dot(z, w1_ref[...], preferred_element_type=jnp.float32)
        + b1_ref[...], 0.0)
    z = jnp.maximum(
        jnp.dot(z, w2_ref[...], preferred_element_type=jnp.float32)
        + b2_ref[...], 0.0)
    z = _bn(z, g1_ref[...], be1_ref[...])
    z = _bn(z, g2_ref[...], be2_ref[...])
    out_ref[...] = z


_dense = pl.pallas_call(
    _dense_body,
    out_shape=jax.ShapeDtypeStruct((N, H), jnp.float32),
)


def _head_body(h_ref, batch_ref, w1_ref, b1_ref, w2_ref, b2_ref, out_ref):
    h = h_ref[...]
    b = batch_ref[...]
    gids = lax.broadcasted_iota(jnp.int32, (N, G), 1)
    onehot = jnp.where(b == gids, 1.0, 0.0)
    sums = lax.dot_general(onehot, h, (((0,), (0,)), ((), ())),
                           preferred_element_type=jnp.float32)
    counts = jnp.sum(onehot, axis=0)[:, None]
    pooled = sums / jnp.maximum(counts, 1.0)
    z = jnp.maximum(
        jnp.dot(pooled, w1_ref[...], preferred_element_type=jnp.float32)
        + b1_ref[...], 0.0)
    z = (jnp.dot(z, w2_ref[...], preferred_element_type=jnp.float32)
         + b2_ref[...])
    m = jnp.max(z, axis=-1, keepdims=True)
    lse = jnp.log(jnp.sum(jnp.exp(z - m), axis=-1, keepdims=True))
    out_ref[...] = z - m - lse


_head = pl.pallas_call(
    _head_body,
    out_shape=jax.ShapeDtypeStruct((G, C), jnp.float32),
)


def kernel(x, edge_index, batch, params):
    src = edge_index[0].astype(jnp.int32)
    dst = edge_index[1].astype(jnp.int32)
    zeros = jnp.zeros((N, H), jnp.float32)
    batch2d = batch.astype(jnp.int32).reshape(N, 1)

    h = x
    for l in range(3):
        p = params['convs'][l]
        bn = params['bns'][l]
        partials = _get_segsum()(h, src, dst, zeros)
        h = _dense(h, partials,
                   p['eps'].reshape(1, 1),
                   p['W1'], p['b1'].reshape(1, H),
                   p['W2'], p['b2'].reshape(1, H),
                   p['g'].reshape(1, H), p['be'].reshape(1, H),
                   bn['g'].reshape(1, H), bn['be'].reshape(1, H))

    logits = _head(h, batch2d,
                   params['lin1_W'], params['lin1_b'].reshape(1, H),
                   params['lin2_W'], params['lin2_b'].reshape(1, C))
    bit_sum = jnp.zeros((1,), jnp.float32)
    return (logits, bit_sum)


# speedup vs baseline: 11.4578x; 1.3225x over previous
import functools

import jax
import jax.numpy as jnp
from jax import lax
from jax.experimental import pallas as pl
from jax.experimental.pallas import tpu as pltpu
from jax.experimental.pallas import tpu_sc as plsc

N = 10000
E = 320000
H = 128
C = 10
G = 128
BN_EPS = 1e-5

NC = 2
NS = 16
NW = NC * NS

CHUNK = 128
TOTAL_CHUNKS = E // CHUNK
MAIN_ITERS = TOTAL_CHUNKS // NW
TAIL_CHUNKS = TOTAL_CHUNKS - MAIN_ITERS * NW
ROWS_PER_TILE = 624
TAIL_ROWS = N - NS * ROWS_PER_TILE


UNROLL = 3
RING = 3
DPRE = 2


def _segsum_body(h_hbm, src_hbm, dst_hbm, zeros_hbm,
                 out_hbm, src_v, dst_v, rows_v, acc_sh,
                 g_sems, s_sems, si_sems, di_sems):
    cid = lax.axis_index("c")
    sid = lax.axis_index("s")
    wid = sid * NC + cid

    def fire_src(i, q):
        base = (wid * MAIN_ITERS + i) * CHUNK
        pltpu.async_copy(src_hbm.at[pl.ds(base, CHUNK)], src_v.at[q],
                         si_sems.at[q])

    def wait_src(i, q):
        base = (wid * MAIN_ITERS + i) * CHUNK
        pltpu.make_async_copy(src_hbm.at[pl.ds(base, CHUNK)], src_v.at[q],
                              si_sems.at[q]).wait()

    def fire_dst(i, q):
        base = (wid * MAIN_ITERS + i) * CHUNK
        pltpu.async_copy(dst_hbm.at[pl.ds(base, CHUNK)], dst_v.at[q],
                         di_sems.at[q])

    def wait_dst(i, q):
        base = (wid * MAIN_ITERS + i) * CHUNK
        pltpu.make_async_copy(dst_hbm.at[pl.ds(base, CHUNK)], dst_v.at[q],
                              di_sems.at[q]).wait()

    def fire_gather(b, q):
        pltpu.async_copy(h_hbm.at[src_v.at[q]], rows_v.at[b], g_sems.at[b])

    def wait_gather(b, q):
        pltpu.make_async_copy(h_hbm.at[src_v.at[q]], rows_v.at[b],
                              g_sems.at[b]).wait()

    def fire_scatter(b, q):
        pltpu.async_copy(rows_v.at[b], acc_sh.at[dst_v.at[q]], s_sems.at[b],
                         add=True)

    def wait_scatter(b, q):
        pltpu.make_async_copy(rows_v.at[b], acc_sh.at[dst_v.at[q]],
                              s_sems.at[b]).wait()

    for q in range(DPRE):
        fire_src(q, q)
        fire_dst(q, q)

    row0 = sid * ROWS_PER_TILE
    pltpu.sync_copy(zeros_hbm.at[pl.ds(row0, ROWS_PER_TILE)],
                    acc_sh.at[pl.ds(row0, ROWS_PER_TILE)])

    @pl.when(sid == NS - 1)
    def _():
        pltpu.sync_copy(zeros_hbm.at[pl.ds(NS * ROWS_PER_TILE, TAIL_ROWS)],
                        acc_sh.at[pl.ds(NS * ROWS_PER_TILE, TAIL_ROWS)])

    wait_src(0, 0)
    fire_gather(0, 0)
    plsc.subcore_barrier()

    def body(it, _):
        for k in range(UNROLL):
            i = it * UNROLL + k
            nk = (k + 1) % RING

            if k == 0:
                @pl.when(it > 0)
                def _():
                    wait_scatter((k - 1) % RING, (k - 1) % RING)
            else:
                wait_scatter(k - 1, k - 1)

            @pl.when(i + DPRE < MAIN_ITERS)
            def _():
                fire_src(i + DPRE, (k + DPRE) % RING)
                fire_dst(i + DPRE, (k + DPRE) % RING)

            @pl.when(i + 1 < MAIN_ITERS)
            def _():
                wait_src(i + 1, nk)
                fire_gather(nk, nk)

            wait_gather(k, k)
            wait_dst(i, k)
            fire_scatter(k, k)
        return ()

    lax.fori_loop(0, MAIN_ITERS // UNROLL, body, (), unroll=False)
    wait_scatter((MAIN_ITERS - 1) % RING, (MAIN_ITERS - 1) % RING)

    @pl.when(wid < TAIL_CHUNKS)
    def _():
        base = (MAIN_ITERS * NW + wid) * CHUNK
        pltpu.sync_copy(src_hbm.at[pl.ds(base, CHUNK)], src_v.at[0])
        pltpu.sync_copy(dst_hbm.at[pl.ds(base, CHUNK)], dst_v.at[0])
        pltpu.async_copy(h_hbm.at[src_v.at[0]], rows_v.at[0], g_sems.at[0]).wait()
        pltpu.sync_copy(rows_v.at[0], acc_sh.at[dst_v.at[0]], add=True)

    plsc.subcore_barrier()

    pltpu.sync_copy(acc_sh.at[pl.ds(row0, ROWS_PER_TILE)],
                    out_hbm.at[cid].at[pl.ds(row0, ROWS_PER_TILE)])

    @pl.when(sid == NS - 1)
    def _():
        pltpu.sync_copy(acc_sh.at[pl.ds(NS * ROWS_PER_TILE, TAIL_ROWS)],
                        out_hbm.at[cid].at[pl.ds(NS * ROWS_PER_TILE, TAIL_ROWS)])


@functools.cache
def _get_segsum():
    return functools.partial(
        pl.kernel,
        out_type=jax.ShapeDtypeStruct((NC, N, H), jnp.float32),
        mesh=plsc.VectorSubcoreMesh(core_axis_name="c", subcore_axis_name="s",
                                    num_cores=NC, num_subcores=NS),
        scratch_types=[
            pltpu.VMEM((RING, CHUNK), jnp.int32),
            pltpu.VMEM((RING, CHUNK), jnp.int32),
            pltpu.VMEM((RING, CHUNK, H), jnp.float32),
            pltpu.VMEM_SHARED((N, H), jnp.float32),
            pltpu.SemaphoreType.DMA((RING,)),
            pltpu.SemaphoreType.DMA((RING,)),
            pltpu.SemaphoreType.DMA((RING,)),
            pltpu.SemaphoreType.DMA((RING,)),
        ],
    )(_segsum_body)


def _bn(z, g, b):
    m = jnp.mean(z, axis=0, keepdims=True)
    v = jnp.mean((z - m) * (z - m), axis=0, keepdims=True)
    return (z - m) * lax.rsqrt(v + BN_EPS) * g + b


def _dense_body(h_ref, a_ref, eps_ref, w1_ref, b1_ref, w2_ref, b2_ref,
                g1_ref, be1_ref, g2_ref, be2_ref, out_ref):
    h = h_ref[...]
    aggr = a_ref[0] + a_ref[1]
    z = (1.0 + eps_ref[0, 0]) * h + aggr
    z = jnp.maximum(
        jnp.dot(z, w1_ref[...], preferred_element_type=jnp.float32)
        + b1_ref[...], 0.0)
    z = jnp.maximum(
        jnp.dot(z, w2_ref[...], preferred_element_type=jnp.float32)
        + b2_ref[...], 0.0)
    z = _bn(z, g1_ref[...], be1_ref[...])
    z = _bn(z, g2_ref[...], be2_ref[...])
    out_ref[...] = z


_dense = pl.pallas_call(
    _dense_body,
    out_shape=jax.ShapeDtypeStruct((N, H), jnp.float32),
)


def _head_body(h_ref, batch_ref, w1_ref, b1_ref, w2_ref, b2_ref, out_ref):
    h = h_ref[...]
    b = batch_ref[...]
    gids = lax.broadcasted_iota(jnp.int32, (N, G), 1)
    onehot = jnp.where(b == gids, 1.0, 0.0)
    sums = lax.dot_general(onehot, h, (((0,), (0,)), ((), ())),
                           preferred_element_type=jnp.float32)
    counts = jnp.sum(onehot, axis=0)[:, None]
    pooled = sums / jnp.maximum(counts, 1.0)
    z = jnp.maximum(
        jnp.dot(pooled, w1_ref[...], preferred_element_type=jnp.float32)
        + b1_ref[...], 0.0)
    z = (jnp.dot(z, w2_ref[...], preferred_element_type=jnp.float32)
         + b2_ref[...])
    m = jnp.max(z, axis=-1, keepdims=True)
    lse = jnp.log(jnp.sum(jnp.exp(z - m), axis=-1, keepdims=True))
    out_ref[...] = z - m - lse


_head = pl.pallas_call(
    _head_body,
    out_shape=jax.ShapeDtypeStruct((G, C), jnp.float32),
)


def kernel(x, edge_index, batch, params):
    src = edge_index[0].astype(jnp.int32)
    dst = edge_index[1].astype(jnp.int32)
    zeros = jnp.zeros((N, H), jnp.float32)
    batch2d = batch.astype(jnp.int32).reshape(N, 1)

    h = x
    for l in range(3):
        p = params['convs'][l]
        bn = params['bns'][l]
        partials = _get_segsum()(h, src, dst, zeros)
        h = _dense(h, partials,
                   p['eps'].reshape(1, 1),
                   p['W1'], p['b1'].reshape(1, H),
                   p['W2'], p['b2'].reshape(1, H),
                   p['g'].reshape(1, H), p['be'].reshape(1, H),
                   bn['g'].reshape(1, H), bn['be'].reshape(1, H))

    logits = _head(h, batch2d,
                   params['lin1_W'], params['lin1_b'].reshape(1, H),
                   params['lin2_W'], params['lin2_b'].reshape(1, C))
    bit_sum = jnp.zeros((1,), jnp.float32)
    return (logits, bit_sum)


# speedup vs baseline: 11.6272x; 1.0148x over previous
import functools

import jax
import jax.numpy as jnp
from jax import lax
from jax.experimental import pallas as pl
from jax.experimental.pallas import tpu as pltpu
from jax.experimental.pallas import tpu_sc as plsc

N = 10000
E = 320000
H = 128
C = 10
G = 128
BN_EPS = 1e-5

NC = 2
NS = 16
NW = NC * NS

CHUNK = 128
TOTAL_CHUNKS = E // CHUNK
MAIN_ITERS = TOTAL_CHUNKS // NW
TAIL_CHUNKS = TOTAL_CHUNKS - MAIN_ITERS * NW
ROWS_PER_TILE = 624
TAIL_ROWS = N - NS * ROWS_PER_TILE


UNROLL = 3
RING = 3
DPRE = 2


def _segsum_body(h_hbm, src_hbm, dst_hbm, zeros_hbm,
                 out_hbm, src_v, dst_v, rows_v, acc_sh,
                 g_sems, s_sems, si_sems, di_sems):
    cid = lax.axis_index("c")
    sid = lax.axis_index("s")
    wid = sid * NC + cid

    def fire_src(i, q):
        base = (wid * MAIN_ITERS + i) * CHUNK
        pltpu.async_copy(src_hbm.at[pl.ds(base, CHUNK)], src_v.at[q],
                         si_sems.at[q])

    def wait_src(i, q):
        base = (wid * MAIN_ITERS + i) * CHUNK
        pltpu.make_async_copy(src_hbm.at[pl.ds(base, CHUNK)], src_v.at[q],
                              si_sems.at[q]).wait()

    def fire_dst(i, q):
        base = (wid * MAIN_ITERS + i) * CHUNK
        pltpu.async_copy(dst_hbm.at[pl.ds(base, CHUNK)], dst_v.at[q],
                         di_sems.at[q])

    def wait_dst(i, q):
        base = (wid * MAIN_ITERS + i) * CHUNK
        pltpu.make_async_copy(dst_hbm.at[pl.ds(base, CHUNK)], dst_v.at[q],
                              di_sems.at[q]).wait()

    def fire_gather(b, q):
        pltpu.async_copy(h_hbm.at[src_v.at[q]], rows_v.at[b], g_sems.at[b])

    def wait_gather(b, q):
        pltpu.make_async_copy(h_hbm.at[src_v.at[q]], rows_v.at[b],
                              g_sems.at[b]).wait()

    def fire_scatter(b, q):
        pltpu.async_copy(rows_v.at[b], acc_sh.at[dst_v.at[q]], s_sems.at[b],
                         add=True)

    def wait_scatter(b, q):
        pltpu.make_async_copy(rows_v.at[b], acc_sh.at[dst_v.at[q]],
                              s_sems.at[b]).wait()

    for q in range(DPRE):
        fire_src(q, q)
        fire_dst(q, q)

    row0 = sid * ROWS_PER_TILE
    pltpu.sync_copy(zeros_hbm.at[pl.ds(row0, ROWS_PER_TILE)],
                    acc_sh.at[pl.ds(row0, ROWS_PER_TILE)])

    @pl.when(sid == NS - 1)
    def _():
        pltpu.sync_copy(zeros_hbm.at[pl.ds(NS * ROWS_PER_TILE, TAIL_ROWS)],
                        acc_sh.at[pl.ds(NS * ROWS_PER_TILE, TAIL_ROWS)])

    wait_src(0, 0)
    fire_gather(0, 0)
    plsc.subcore_barrier()

    def body(it, _):
        for k in range(UNROLL):
            i = it * UNROLL + k
            nk = (k + 1) % RING

            if k == 0:
                @pl.when(it > 0)
                def _():
                    wait_scatter((k - 1) % RING, (k - 1) % RING)
            else:
                wait_scatter(k - 1, k - 1)

            @pl.when(i + DPRE < MAIN_ITERS)
            def _():
                fire_src(i + DPRE, (k + DPRE) % RING)
                fire_dst(i + DPRE, (k + DPRE) % RING)

            @pl.when(i + 1 < MAIN_ITERS)
            def _():
                wait_src(i + 1, nk)
                fire_gather(nk, nk)

            wait_gather(k, k)
            wait_dst(i, k)
            fire_scatter(k, k)
        return ()

    lax.fori_loop(0, MAIN_ITERS // UNROLL, body, (), unroll=False)
    wait_scatter((MAIN_ITERS - 1) % RING, (MAIN_ITERS - 1) % RING)

    @pl.when(wid < TAIL_CHUNKS)
    def _():
        base = (MAIN_ITERS * NW + wid) * CHUNK
        pltpu.sync_copy(src_hbm.at[pl.ds(base, CHUNK)], src_v.at[0])
        pltpu.sync_copy(dst_hbm.at[pl.ds(base, CHUNK)], dst_v.at[0])
        pltpu.async_copy(h_hbm.at[src_v.at[0]], rows_v.at[0], g_sems.at[0]).wait()
        pltpu.sync_copy(rows_v.at[0], acc_sh.at[dst_v.at[0]], add=True)

    plsc.subcore_barrier()

    pltpu.sync_copy(acc_sh.at[pl.ds(row0, ROWS_PER_TILE)],
                    out_hbm.at[cid].at[pl.ds(row0, ROWS_PER_TILE)])

    @pl.when(sid == NS - 1)
    def _():
        pltpu.sync_copy(acc_sh.at[pl.ds(NS * ROWS_PER_TILE, TAIL_ROWS)],
                        out_hbm.at[cid].at[pl.ds(NS * ROWS_PER_TILE, TAIL_ROWS)])


@functools.cache
def _get_segsum():
    return functools.partial(
        pl.kernel,
        out_type=jax.ShapeDtypeStruct((NC, N, H), jnp.float32),
        mesh=plsc.VectorSubcoreMesh(core_axis_name="c", subcore_axis_name="s",
                                    num_cores=NC, num_subcores=NS),
        scratch_types=[
            pltpu.VMEM((RING, CHUNK), jnp.int32),
            pltpu.VMEM((RING, CHUNK), jnp.int32),
            pltpu.VMEM((RING, CHUNK, H), jnp.float32),
            pltpu.VMEM_SHARED((N, H), jnp.float32),
            pltpu.SemaphoreType.DMA((RING,)),
            pltpu.SemaphoreType.DMA((RING,)),
            pltpu.SemaphoreType.DMA((RING,)),
            pltpu.SemaphoreType.DMA((RING,)),
        ],
    )(_segsum_body)


def _bn(z, g, b):
    m = jnp.mean(z, axis=0, keepdims=True)
    v = jnp.mean((z - m) * (z - m), axis=0, keepdims=True)
    return (z - m) * lax.rsqrt(v + BN_EPS) * g + b


def _dense_body(h_ref, a_ref, eps_ref, w1_ref, b1_ref, w2_ref, b2_ref,
                g1_ref, be1_ref, g2_ref, be2_ref, out_ref):
    h = h_ref[...]
    aggr = a_ref[0] + a_ref[1]
    z = (1.0 + eps_ref[0, 0]) * h + aggr
    z = jnp.maximum(
        jnp.dot(z, w1_ref[...], preferred_element_type=jnp.float32)
        + b1_ref[...], 0.0)
    z = jnp.maximum(
        jnp.dot(z, w2_ref[...], preferred_element_type=jnp.float32)
        + b2_ref[...], 0.0)
    z = _bn(z, g1_ref[...], be1_ref[...])
    z = _bn(z, g2_ref[...], be2_ref[...])
    out_ref[...] = z


_dense = pl.pallas_call(
    _dense_body,
    out_shape=jax.ShapeDtypeStruct((N, H), jnp.float32),
)


def _dense_head_body(h_ref, a_ref, eps_ref, w1_ref, b1_ref, w2_ref, b2_ref,
                     g1_ref, be1_ref, g2_ref, be2_ref, batch_ref,
                     l1w_ref, l1b_ref, l2w_ref, l2b_ref, out_ref):
    h = h_ref[...]
    aggr = a_ref[0] + a_ref[1]
    z = (1.0 + eps_ref[0, 0]) * h + aggr
    z = jnp.maximum(
        jnp.dot(z, w1_ref[...], preferred_element_type=jnp.float32)
        + b1_ref[...], 0.0)
    z = jnp.maximum(
        jnp.dot(z, w2_ref[...], preferred_element_type=jnp.float32)
        + b2_ref[...], 0.0)
    z = _bn(z, g1_ref[...], be1_ref[...])
    h3 = _bn(z, g2_ref[...], be2_ref[...])

    b = batch_ref[...]
    gids = lax.broadcasted_iota(jnp.int32, (N, G), 1)
    onehot = jnp.where(b == gids, 1.0, 0.0)
    sums = lax.dot_general(onehot, h3, (((0,), (0,)), ((), ())),
                           preferred_element_type=jnp.float32)
    counts = jnp.sum(onehot, axis=0)[:, None]
    pooled = sums / jnp.maximum(counts, 1.0)
    z = jnp.maximum(
        jnp.dot(pooled, l1w_ref[...], preferred_element_type=jnp.float32)
        + l1b_ref[...], 0.0)
    z = (jnp.dot(z, l2w_ref[...], preferred_element_type=jnp.float32)
         + l2b_ref[...])
    m = jnp.max(z, axis=-1, keepdims=True)
    lse = jnp.log(jnp.sum(jnp.exp(z - m), axis=-1, keepdims=True))
    out_ref[...] = z - m - lse


_dense_head = pl.pallas_call(
    _dense_head_body,
    out_shape=jax.ShapeDtypeStruct((G, C), jnp.float32),
)


def _head_body(h_ref, batch_ref, w1_ref, b1_ref, w2_ref, b2_ref, out_ref):
    h = h_ref[...]
    b = batch_ref[...]
    gids = lax.broadcasted_iota(jnp.int32, (N, G), 1)
    onehot = jnp.where(b == gids, 1.0, 0.0)
    sums = lax.dot_general(onehot, h, (((0,), (0,)), ((), ())),
                           preferred_element_type=jnp.float32)
    counts = jnp.sum(onehot, axis=0)[:, None]
    pooled = sums / jnp.maximum(counts, 1.0)
    z = jnp.maximum(
        jnp.dot(pooled, w1_ref[...], preferred_element_type=jnp.float32)
        + b1_ref[...], 0.0)
    z = (jnp.dot(z, w2_ref[...], preferred_element_type=jnp.float32)
         + b2_ref[...])
    m = jnp.max(z, axis=-1, keepdims=True)
    lse = jnp.log(jnp.sum(jnp.exp(z - m), axis=-1, keepdims=True))
    out_ref[...] = z - m - lse


_head = pl.pallas_call(
    _head_body,
    out_shape=jax.ShapeDtypeStruct((G, C), jnp.float32),
)


def kernel(x, edge_index, batch, params):
    src = edge_index[0].astype(jnp.int32)
    dst = edge_index[1].astype(jnp.int32)
    zeros = jnp.zeros((N, H), jnp.float32)
    batch2d = batch.astype(jnp.int32).reshape(N, 1)

    h = x
    for l in range(2):
        p = params['convs'][l]
        bn = params['bns'][l]
        partials = _get_segsum()(h, src, dst, zeros)
        h = _dense(h, partials,
                   p['eps'].reshape(1, 1),
                   p['W1'], p['b1'].reshape(1, H),
                   p['W2'], p['b2'].reshape(1, H),
                   p['g'].reshape(1, H), p['be'].reshape(1, H),
                   bn['g'].reshape(1, H), bn['be'].reshape(1, H))

    p = params['convs'][2]
    bn = params['bns'][2]
    partials = _get_segsum()(h, src, dst, zeros)
    logits = _dense_head(h, partials,
                         p['eps'].reshape(1, 1),
                         p['W1'], p['b1'].reshape(1, H),
                         p['W2'], p['b2'].reshape(1, H),
                         p['g'].reshape(1, H), p['be'].reshape(1, H),
                         bn['g'].reshape(1, H), bn['be'].reshape(1, H),
                         batch2d,
                         params['lin1_W'], params['lin1_b'].reshape(1, H),
                         params['lin2_W'], params['lin2_b'].reshape(1, C))
    bit_sum = jnp.zeros((1,), jnp.float32)
    return (logits, bit_sum)


# speedup vs baseline: 11.6409x; 1.0012x over previous
import functools

import jax
import jax.numpy as jnp
from jax import lax
from jax.experimental import pallas as pl
from jax.experimental.pallas import tpu as pltpu
from jax.experimental.pallas import tpu_sc as plsc

N = 10000
E = 320000
H = 128
C = 10
G = 128
BN_EPS = 1e-5

NC = 2
NS = 16
NW = NC * NS

CHUNK = 128
TOTAL_CHUNKS = E // CHUNK
MAIN_ITERS = TOTAL_CHUNKS // NW
TAIL_CHUNKS = TOTAL_CHUNKS - MAIN_ITERS * NW
ROWS_PER_TILE = 624
TAIL_ROWS = N - NS * ROWS_PER_TILE


UNROLL = 3
RING = 3
DPRE = 2


def _segsum_body(h_hbm, src_hbm, dst_hbm, zeros_hbm,
                 out_hbm, src_v, dst_v, rows_v, acc_sh,
                 g_sems, s_sems, si_sems, di_sems):
    cid = lax.axis_index("c")
    sid = lax.axis_index("s")
    wid = sid * NC + cid

    def fire_src(i, q):
        base = (wid * MAIN_ITERS + i) * CHUNK
        pltpu.async_copy(src_hbm.at[pl.ds(base, CHUNK)], src_v.at[q],
                         si_sems.at[q])

    def wait_src(i, q):
        base = (wid * MAIN_ITERS + i) * CHUNK
        pltpu.make_async_copy(src_hbm.at[pl.ds(base, CHUNK)], src_v.at[q],
                              si_sems.at[q]).wait()

    def fire_dst(i, q):
        base = (wid * MAIN_ITERS + i) * CHUNK
        pltpu.async_copy(dst_hbm.at[pl.ds(base, CHUNK)], dst_v.at[q],
                         di_sems.at[q])

    def wait_dst(i, q):
        base = (wid * MAIN_ITERS + i) * CHUNK
        pltpu.make_async_copy(dst_hbm.at[pl.ds(base, CHUNK)], dst_v.at[q],
                              di_sems.at[q]).wait()

    def fire_gather(b, q):
        pltpu.async_copy(h_hbm.at[src_v.at[q]], rows_v.at[b], g_sems.at[b])

    def wait_gather(b, q):
        pltpu.make_async_copy(h_hbm.at[src_v.at[q]], rows_v.at[b],
                              g_sems.at[b]).wait()

    def fire_scatter(b, q):
        pltpu.async_copy(rows_v.at[b], acc_sh.at[dst_v.at[q]], s_sems.at[b],
                         add=True)

    def wait_scatter(b, q):
        pltpu.make_async_copy(rows_v.at[b], acc_sh.at[dst_v.at[q]],
                              s_sems.at[b]).wait()

    for q in range(DPRE):
        fire_src(q, q)
        fire_dst(q, q)

    row0 = sid * ROWS_PER_TILE
    pltpu.sync_copy(zeros_hbm.at[pl.ds(row0, ROWS_PER_TILE)],
                    acc_sh.at[pl.ds(row0, ROWS_PER_TILE)])

    @pl.when(sid == NS - 1)
    def _():
        pltpu.sync_copy(zeros_hbm.at[pl.ds(NS * ROWS_PER_TILE, TAIL_ROWS)],
                        acc_sh.at[pl.ds(NS * ROWS_PER_TILE, TAIL_ROWS)])

    wait_src(0, 0)
    fire_gather(0, 0)
    plsc.subcore_barrier()

    def body(it, _):
        for k in range(UNROLL):
            i = it * UNROLL + k
            nk = (k + 1) % RING

            if k == 0:
                @pl.when(it > 0)
                def _():
                    wait_scatter((k - 1) % RING, (k - 1) % RING)
            else:
                wait_scatter(k - 1, k - 1)

            @pl.when(i + DPRE < MAIN_ITERS)
            def _():
                fire_src(i + DPRE, (k + DPRE) % RING)
                fire_dst(i + DPRE, (k + DPRE) % RING)

            @pl.when(i + 1 < MAIN_ITERS)
            def _():
                wait_src(i + 1, nk)
                fire_gather(nk, nk)

            wait_gather(k, k)
            wait_dst(i, k)
            fire_scatter(k, k)
        return ()

    lax.fori_loop(0, MAIN_ITERS // UNROLL, body, (), unroll=False)
    wait_scatter((MAIN_ITERS - 1) % RING, (MAIN_ITERS - 1) % RING)

    @pl.when(wid < TAIL_CHUNKS)
    def _():
        base = (MAIN_ITERS * NW + wid) * CHUNK
        pltpu.sync_copy(src_hbm.at[pl.ds(base, CHUNK)], src_v.at[0])
        pltpu.sync_copy(dst_hbm.at[pl.ds(base, CHUNK)], dst_v.at[0])
        pltpu.async_copy(h_hbm.at[src_v.at[0]], rows_v.at[0], g_sems.at[0]).wait()
        pltpu.sync_copy(rows_v.at[0], acc_sh.at[dst_v.at[0]], add=True)

    plsc.subcore_barrier()

    pltpu.sync_copy(acc_sh.at[pl.ds(row0, ROWS_PER_TILE)],
                    out_hbm.at[cid].at[pl.ds(row0, ROWS_PER_TILE)])

    @pl.when(sid == NS - 1)
    def _():
        pltpu.sync_copy(acc_sh.at[pl.ds(NS * ROWS_PER_TILE, TAIL_ROWS)],
                        out_hbm.at[cid].at[pl.ds(NS * ROWS_PER_TILE, TAIL_ROWS)])


@functools.cache
def _get_segsum():
    return functools.partial(
        pl.kernel,
        out_type=jax.ShapeDtypeStruct((NC, N, H), jnp.float32),
        mesh=plsc.VectorSubcoreMesh(core_axis_name="c", subcore_axis_name="s",
                                    num_cores=NC, num_subcores=NS),
        scratch_types=[
            pltpu.VMEM((RING, CHUNK), jnp.int32),
            pltpu.VMEM((RING, CHUNK), jnp.int32),
            pltpu.VMEM((RING, CHUNK, H), jnp.float32),
            pltpu.VMEM_SHARED((N, H), jnp.float32),
            pltpu.SemaphoreType.DMA((RING,)),
            pltpu.SemaphoreType.DMA((RING,)),
            pltpu.SemaphoreType.DMA((RING,)),
            pltpu.SemaphoreType.DMA((RING,)),
        ],
    )(_segsum_body)


def _bn(z, g, b):
    m = jnp.mean(z, axis=0, keepdims=True)
    v = jnp.mean((z - m) * (z - m), axis=0, keepdims=True)
    return (z - m) * lax.rsqrt(v + BN_EPS) * g + b


def _dense_body(h_ref, a_ref, eps_ref, w1_ref, b1_ref, w2_ref, b2_ref,
                g1_ref, be1_ref, g2_ref, be2_ref, out_ref):
    h = h_ref[...]
    aggr = a_ref[0] + a_ref[1]
    z = (1.0 + eps_ref[0, 0]) * h + aggr
    z = jnp.maximum(
        jnp.dot(z, w1_ref[...], preferred_element_type=jnp.float32)
        + b1_ref[...], 0.0)
    z = jnp.maximum(
        jnp.dot(z, w2_ref[...], preferred_element_type=jnp.float32)
        + b2_ref[...], 0.0)
    z = _bn(z, g1_ref[...], be1_ref[...])
    z = _bn(z, g2_ref[...], be2_ref[...])
    out_ref[...] = z


_dense = pl.pallas_call(
    _dense_body,
    out_shape=jax.ShapeDtypeStruct((N, H), jnp.float32),
)


def _dense_head_body(h_ref, a_ref, eps_ref, w1_ref, b1_ref, w2_ref, b2_ref,
                     g1_ref, be1_ref, g2_ref, be2_ref, batch_ref,
                     l1w_ref, l1b_ref, l2w_ref, l2b_ref, out_ref):
    h = h_ref[...]
    aggr = a_ref[0] + a_ref[1]
    z = (1.0 + eps_ref[0, 0]) * h + aggr
    z = jnp.maximum(
        jnp.dot(z, w1_ref[...], preferred_element_type=jnp.float32)
        + b1_ref[...], 0.0)
    z = jnp.maximum(
        jnp.dot(z, w2_ref[...], preferred_element_type=jnp.float32)
        + b2_ref[...], 0.0)
    z = _bn(z, g1_ref[...], be1_ref[...])
    h3 = _bn(z, g2_ref[...], be2_ref[...])

    b = batch_ref[...]
    gids = lax.broadcasted_iota(jnp.int32, (N, G), 1)
    onehot = jnp.where(b == gids, 1.0, 0.0)
    sums = lax.dot_general(onehot, h3, (((0,), (0,)), ((), ())),
                           preferred_element_type=jnp.float32)
    counts = jnp.sum(onehot, axis=0)[:, None]
    pooled = sums / jnp.maximum(counts, 1.0)
    z = jnp.maximum(
        jnp.dot(pooled, l1w_ref[...], preferred_element_type=jnp.float32)
        + l1b_ref[...], 0.0)
    z = (jnp.dot(z, l2w_ref[...], preferred_element_type=jnp.float32)
         + l2b_ref[...])
    m = jnp.max(z, axis=-1, keepdims=True)
    lse = jnp.log(jnp.sum(jnp.exp(z - m), axis=-1, keepdims=True))
    out_ref[...] = z - m - lse


_dense_head = pl.pallas_call(
    _dense_head_body,
    out_shape=jax.ShapeDtypeStruct((G, C), jnp.float32),
)


def kernel(x, edge_index, batch, params):
    src = edge_index[0].astype(jnp.int32)
    dst = edge_index[1].astype(jnp.int32)
    zeros = jnp.zeros((N, H), jnp.float32)
    batch2d = batch.astype(jnp.int32).reshape(N, 1)

    h = x
    for l in range(2):
        p = params['convs'][l]
        bn = params['bns'][l]
        partials = _get_segsum()(h, src, dst, zeros)
        h = _dense(h, partials,
                   p['eps'].reshape(1, 1),
                   p['W1'], p['b1'].reshape(1, H),
                   p['W2'], p['b2'].reshape(1, H),
                   p['g'].reshape(1, H), p['be'].reshape(1, H),
                   bn['g'].reshape(1, H), bn['be'].reshape(1, H))

    p = params['convs'][2]
    bn = params['bns'][2]
    partials = _get_segsum()(h, src, dst, zeros)
    logits = _dense_head(h, partials,
                         p['eps'].reshape(1, 1),
                         p['W1'], p['b1'].reshape(1, H),
                         p['W2'], p['b2'].reshape(1, H),
                         p['g'].reshape(1, H), p['be'].reshape(1, H),
                         bn['g'].reshape(1, H), bn['be'].reshape(1, H),
                         batch2d,
                         params['lin1_W'], params['lin1_b'].reshape(1, H),
                         params['lin2_W'], params['lin2_b'].reshape(1, C))
    bit_sum = jnp.zeros((1,), jnp.float32)
    return (logits, bit_sum)


# speedup vs baseline: 11.7138x; 1.0063x over previous
import functools

import jax
import jax.numpy as jnp
from jax import lax
from jax.experimental import pallas as pl
from jax.experimental.pallas import tpu as pltpu
from jax.experimental.pallas import tpu_sc as plsc

N = 10000
E = 320000
H = 128
C = 10
G = 128
BN_EPS = 1e-5

NC = 2
NS = 16
NW = NC * NS

CHUNK = 128
TOTAL_CHUNKS = E // CHUNK
MAIN_ITERS = TOTAL_CHUNKS // NW
TAIL_CHUNKS = TOTAL_CHUNKS - MAIN_ITERS * NW
ROWS_PER_TILE = 624
TAIL_ROWS = N - NS * ROWS_PER_TILE


UNROLL = 3
RING = 3
DPRE = 2


def _segsum_body(h_hbm, src_hbm, dst_hbm, zeros_hbm,
                 out_hbm, src_v, dst_v, rows_v, acc_sh,
                 g_sems, s_sems, si_sems, di_sems, z_sem):
    cid = lax.axis_index("c")
    sid = lax.axis_index("s")
    wid = sid * NC + cid

    def fire_src(i, q):
        base = (wid * MAIN_ITERS + i) * CHUNK
        pltpu.async_copy(src_hbm.at[pl.ds(base, CHUNK)], src_v.at[q],
                         si_sems.at[q])

    def wait_src(i, q):
        base = (wid * MAIN_ITERS + i) * CHUNK
        pltpu.make_async_copy(src_hbm.at[pl.ds(base, CHUNK)], src_v.at[q],
                              si_sems.at[q]).wait()

    def fire_dst(i, q):
        base = (wid * MAIN_ITERS + i) * CHUNK
        pltpu.async_copy(dst_hbm.at[pl.ds(base, CHUNK)], dst_v.at[q],
                         di_sems.at[q])

    def wait_dst(i, q):
        base = (wid * MAIN_ITERS + i) * CHUNK
        pltpu.make_async_copy(dst_hbm.at[pl.ds(base, CHUNK)], dst_v.at[q],
                              di_sems.at[q]).wait()

    def fire_gather(b, q):
        pltpu.async_copy(h_hbm.at[src_v.at[q]], rows_v.at[b], g_sems.at[b])

    def wait_gather(b, q):
        pltpu.make_async_copy(h_hbm.at[src_v.at[q]], rows_v.at[b],
                              g_sems.at[b]).wait()

    def fire_scatter(b, q):
        pltpu.async_copy(rows_v.at[b], acc_sh.at[dst_v.at[q]], s_sems.at[b],
                         add=True)

    def wait_scatter(b, q):
        pltpu.make_async_copy(rows_v.at[b], acc_sh.at[dst_v.at[q]],
                              s_sems.at[b]).wait()

    for q in range(DPRE):
        fire_src(q, q)
        fire_dst(q, q)

    row0 = sid * ROWS_PER_TILE
    pltpu.async_copy(zeros_hbm.at[pl.ds(row0, ROWS_PER_TILE)],
                     acc_sh.at[pl.ds(row0, ROWS_PER_TILE)], z_sem)

    @pl.when(sid == NS - 1)
    def _():
        pltpu.sync_copy(zeros_hbm.at[pl.ds(NS * ROWS_PER_TILE, TAIL_ROWS)],
                        acc_sh.at[pl.ds(NS * ROWS_PER_TILE, TAIL_ROWS)])

    wait_src(0, 0)
    fire_gather(0, 0)
    pltpu.make_async_copy(zeros_hbm.at[pl.ds(row0, ROWS_PER_TILE)],
                          acc_sh.at[pl.ds(row0, ROWS_PER_TILE)], z_sem).wait()
    plsc.subcore_barrier()

    def body(it, _):
        for k in range(UNROLL):
            i = it * UNROLL + k
            nk = (k + 1) % RING

            if k == 0:
                @pl.when(it > 0)
                def _():
                    wait_scatter((k - 1) % RING, (k - 1) % RING)
            else:
                wait_scatter(k - 1, k - 1)

            @pl.when(i + DPRE < MAIN_ITERS)
            def _():
                fire_src(i + DPRE, (k + DPRE) % RING)
                fire_dst(i + DPRE, (k + DPRE) % RING)

            @pl.when(i + 1 < MAIN_ITERS)
            def _():
                wait_src(i + 1, nk)
                fire_gather(nk, nk)

            wait_gather(k, k)
            wait_dst(i, k)
            fire_scatter(k, k)
        return ()

    lax.fori_loop(0, MAIN_ITERS // UNROLL, body, (), unroll=False)
    wait_scatter((MAIN_ITERS - 1) % RING, (MAIN_ITERS - 1) % RING)

    @pl.when(wid < TAIL_CHUNKS)
    def _():
        base = (MAIN_ITERS * NW + wid) * CHUNK
        pltpu.sync_copy(src_hbm.at[pl.ds(base, CHUNK)], src_v.at[0])
        pltpu.sync_copy(dst_hbm.at[pl.ds(base, CHUNK)], dst_v.at[0])
        pltpu.async_copy(h_hbm.at[src_v.at[0]], rows_v.at[0], g_sems.at[0]).wait()
        pltpu.sync_copy(rows_v.at[0], acc_sh.at[dst_v.at[0]], add=True)

    plsc.subcore_barrier()

    pltpu.sync_copy(acc_sh.at[pl.ds(row0, ROWS_PER_TILE)],
                    out_hbm.at[cid].at[pl.ds(row0, ROWS_PER_TILE)])

    @pl.when(sid == NS - 1)
    def _():
        pltpu.sync_copy(acc_sh.at[pl.ds(NS * ROWS_PER_TILE, TAIL_ROWS)],
                        out_hbm.at[cid].at[pl.ds(NS * ROWS_PER_TILE, TAIL_ROWS)])


@functools.cache
def _get_segsum():
    return functools.partial(
        pl.kernel,
        out_type=jax.ShapeDtypeStruct((NC, N, H), jnp.float32),
        mesh=plsc.VectorSubcoreMesh(core_axis_name="c", subcore_axis_name="s",
                                    num_cores=NC, num_subcores=NS),
        scratch_types=[
            pltpu.VMEM((RING, CHUNK), jnp.int32),
            pltpu.VMEM((RING, CHUNK), jnp.int32),
            pltpu.VMEM((RING, CHUNK, H), jnp.float32),
            pltpu.VMEM_SHARED((N, H), jnp.float32),
            pltpu.SemaphoreType.DMA((RING,)),
            pltpu.SemaphoreType.DMA((RING,)),
            pltpu.SemaphoreType.DMA((RING,)),
            pltpu.SemaphoreType.DMA((RING,)),
            pltpu.SemaphoreType.DMA,
        ],
    )(_segsum_body)


def _bn(z, g, b):
    m = jnp.mean(z, axis=0, keepdims=True)
    v = jnp.mean((z - m) * (z - m), axis=0, keepdims=True)
    return (z - m) * lax.rsqrt(v + BN_EPS) * g + b


def _dense_body(h_ref, a_ref, eps_ref, w1_ref, b1_ref, w2_ref, b2_ref,
                g1_ref, be1_ref, g2_ref, be2_ref, out_ref):
    h = h_ref[...]
    aggr = a_ref[0] + a_ref[1]
    z = (1.0 + eps_ref[0, 0]) * h + aggr
    z = jnp.maximum(
        jnp.dot(z, w1_ref[...], preferred_element_type=jnp.float32)
        + b1_ref[...], 0.0)
    z = jnp.maximum(
        jnp.dot(z, w2_ref[...], preferred_element_type=jnp.float32)
        + b2_ref[...], 0.0)
    z = _bn(z, g1_ref[...], be1_ref[...])
    z = _bn(z, g2_ref[...], be2_ref[...])
    out_ref[...] = z


_dense = pl.pallas_call(
    _dense_body,
    out_shape=jax.ShapeDtypeStruct((N, H), jnp.float32),
)


def _dense_head_body(h_ref, a_ref, eps_ref, w1_ref, b1_ref, w2_ref, b2_ref,
                     g1_ref, be1_ref, g2_ref, be2_ref, batch_ref,
                     l1w_ref, l1b_ref, l2w_ref, l2b_ref, out_ref):
    h = h_ref[...]
    aggr = a_ref[0] + a_ref[1]
    z = (1.0 + eps_ref[0, 0]) * h + aggr
    z = jnp.maximum(
        jnp.dot(z, w1_ref[...], preferred_element_type=jnp.float32)
        + b1_ref[...], 0.0)
    z = jnp.maximum(
        jnp.dot(z, w2_ref[...], preferred_element_type=jnp.float32)
        + b2_ref[...], 0.0)
    z = _bn(z, g1_ref[...], be1_ref[...])
    h3 = _bn(z, g2_ref[...], be2_ref[...])

    b = batch_ref[...]
    gids = lax.broadcasted_iota(jnp.int32, (N, G), 1)
    onehot = jnp.where(b == gids, 1.0, 0.0)
    sums = lax.dot_general(onehot, h3, (((0,), (0,)), ((), ())),
                           preferred_element_type=jnp.float32)
    counts = jnp.sum(onehot, axis=0)[:, None]
    pooled = sums / jnp.maximum(counts, 1.0)
    z = jnp.maximum(
        jnp.dot(pooled, l1w_ref[...], preferred_element_type=jnp.float32)
        + l1b_ref[...], 0.0)
    z = (jnp.dot(z, l2w_ref[...], preferred_element_type=jnp.float32)
         + l2b_ref[...])
    m = jnp.max(z, axis=-1, keepdims=True)
    lse = jnp.log(jnp.sum(jnp.exp(z - m), axis=-1, keepdims=True))
    out_ref[...] = z - m - lse


_dense_head = pl.pallas_call(
    _dense_head_body,
    out_shape=jax.ShapeDtypeStruct((G, C), jnp.float32),
)


def kernel(x, edge_index, batch, params):
    src = edge_index[0].astype(jnp.int32)
    dst = edge_index[1].astype(jnp.int32)
    zeros = jnp.zeros((N, H), jnp.float32)
    batch2d = batch.astype(jnp.int32).reshape(N, 1)

    h = x
    for l in range(2):
        p = params['convs'][l]
        bn = params['bns'][l]
        partials = _get_segsum()(h, src, dst, zeros)
        h = _dense(h, partials,
                   p['eps'].reshape(1, 1),
                   p['W1'], p['b1'].reshape(1, H),
                   p['W2'], p['b2'].reshape(1, H),
                   p['g'].reshape(1, H), p['be'].reshape(1, H),
                   bn['g'].reshape(1, H), bn['be'].reshape(1, H))

    p = params['convs'][2]
    bn = params['bns'][2]
    partials = _get_segsum()(h, src, dst, zeros)
    logits = _dense_head(h, partials,
                         p['eps'].reshape(1, 1),
                         p['W1'], p['b1'].reshape(1, H),
                         p['W2'], p['b2'].reshape(1, H),
                         p['g'].reshape(1, H), p['be'].reshape(1, H),
                         bn['g'].reshape(1, H), bn['be'].reshape(1, H),
                         batch2d,
                         params['lin1_W'], params['lin1_b'].reshape(1, H),
                         params['lin2_W'], params['lin2_b'].reshape(1, C))
    bit_sum = jnp.zeros((1,), jnp.float32)
    return (logits, bit_sum)
